# Initial kernel scaffold; baseline (speedup 1.0000x reference)
import jax
import jax.numpy as jnp
from jax.experimental import pallas as pl


def kernel(x, edge_index, w_e1, b_e1, w_e2, b_e2, w_efc, b_efc, w_d1, b_d1, w_d2, b_d2, w_dfc, b_dfc):
    raise NotImplementedError("write your pallas kernel here")


# speedup vs baseline: 10.1400x; 10.1400x over previous
import functools

import jax
import jax.numpy as jnp
from jax import lax
from jax.experimental import pallas as pl
from jax.experimental.pallas import tpu as pltpu
from jax.experimental.pallas import tpu_sc as plsc

NN = 10000
EE = 320000
NC, NS, LANES = 2, 16, 16
NW = NC * NS
B = 128
CH_PER = -(-EE // (B * NW))
EPAD = CH_PER * B * NW
NA = 10240
SLAB = NA // NS
DEGW = 16
BM = 1000
GRID = NN // BM

_mesh = plsc.VectorSubcoreMesh(core_axis_name="c", subcore_axis_name="s")


def _edge_loop(wid, body):
    base = wid * CH_PER

    def fbody(j, carry):
        body(base + j)
        return carry

    lax.fori_loop(0, CH_PER, fbody, 0)


def _make_prop(F):
    ZR = 160

    @functools.partial(
        pl.kernel,
        out_type=jax.ShapeDtypeStruct((NC, NA, F), jnp.float32),
        mesh=_mesh,
        scratch_types=[
            pltpu.VMEM((1, B), jnp.int32),
            pltpu.VMEM((B,), jnp.int32),
            pltpu.VMEM((B, F), jnp.float32),
            pltpu.VMEM((ZR, F), jnp.float32),
            pltpu.VMEM_SHARED((NA, F), jnp.float32),
            pltpu.SemaphoreType.DMA,
        ],
        compiler_params=pltpu.CompilerParams(use_tc_tiling_on_sc=False),
    )
    def prop(h_hbm, src_hbm, dst_hbm, out_hbm, dsti_v, srci_v, rows_v, zb_v,
             acc_sh, sem):
        cid = lax.axis_index("c")
        sid = lax.axis_index("s")
        wid = sid * NC + cid

        zero16 = jnp.zeros((LANES,), jnp.float32)

        def zb(r, carry):
            for f in range(F // LANES):
                zb_v[r, pl.ds(f * LANES, LANES)] = zero16
            return carry

        lax.fori_loop(0, ZR, zb, 0)
        for t in range(SLAB // ZR):
            pltpu.sync_copy(zb_v, acc_sh.at[pl.ds(sid * SLAB + t * ZR, ZR)])
        plsc.subcore_barrier()

        def chunk(g):
            off = pl.multiple_of(g * B, B)
            pltpu.sync_copy(src_hbm.at[pl.ds(off, B)], srci_v)
            pltpu.sync_copy(dst_hbm.at[pl.ds(off, B)], dsti_v.at[0])
            pltpu.async_copy(h_hbm.at[srci_v], rows_v, sem).wait()
            pltpu.sync_copy(rows_v, acc_sh.at[dsti_v.at[0]], add=True)

        _edge_loop(wid, chunk)
        plsc.subcore_barrier()

        for t in range(SLAB // ZR):
            row0 = sid * SLAB + t * ZR
            pltpu.sync_copy(acc_sh.at[pl.ds(row0, ZR)], zb_v)
            pltpu.sync_copy(zb_v, out_hbm.at[cid, pl.ds(row0, ZR)])

    return prop


_prop128 = _make_prop(128)
_prop64 = _make_prop(64)


@functools.partial(
    pl.kernel,
    out_type=jax.ShapeDtypeStruct((NC, NA, DEGW), jnp.float32),
    mesh=_mesh,
    scratch_types=[
        pltpu.VMEM((1, B), jnp.int32),
        pltpu.VMEM((B, DEGW), jnp.float32),
        pltpu.VMEM((SLAB, DEGW), jnp.float32),
        pltpu.VMEM_SHARED((NA, DEGW), jnp.float32),
    ],
    compiler_params=pltpu.CompilerParams(use_tc_tiling_on_sc=False),
)
def _sc_degrees(dst_hbm, out_hbm, dsti_v, ones_v, zb_v, acc_sh):
    cid = lax.axis_index("c")
    sid = lax.axis_index("s")
    wid = sid * NC + cid

    one16 = jnp.ones((LANES,), jnp.float32)
    zero16 = jnp.zeros((LANES,), jnp.float32)

    def fill(r, carry):
        ones_v[r] = one16
        return carry

    lax.fori_loop(0, B, fill, 0)

    def zb(r, carry):
        zb_v[r] = zero16
        return carry

    lax.fori_loop(0, SLAB, zb, 0)
    pltpu.sync_copy(zb_v, acc_sh.at[pl.ds(sid * SLAB, SLAB)])
    plsc.subcore_barrier()

    def chunk(g):
        off = pl.multiple_of(g * B, B)
        pltpu.sync_copy(dst_hbm.at[pl.ds(off, B)], dsti_v.at[0])
        pltpu.sync_copy(ones_v, acc_sh.at[dsti_v.at[0]], add=True)

    _edge_loop(wid, chunk)
    plsc.subcore_barrier()

    row0 = sid * SLAB
    pltpu.sync_copy(acc_sh.at[pl.ds(row0, SLAB)], zb_v)
    pltpu.sync_copy(zb_v, out_hbm.at[cid, pl.ds(row0, SLAB)])


def _dot(a, b):
    return jnp.dot(a, b, preferred_element_type=jnp.float32,
                   precision=lax.Precision.HIGHEST)


def _row_spec(f):
    return pl.BlockSpec((BM, f), lambda i: (i, 0))


def _full_spec(shape):
    nd = len(shape)
    return pl.BlockSpec(shape, lambda i, _n=nd: (0,) * _n)


def _part_spec(f):
    return pl.BlockSpec((NC, BM, f), lambda i: (0, i, 0))


def _tc0_body(x_ref, w_ref, o_ref):
    o_ref[...] = _dot(x_ref[...], w_ref[...])


def _tc0(x, w):
    return pl.pallas_call(
        _tc0_body,
        grid=(GRID,),
        in_specs=[_row_spec(128), _full_spec((128, 128))],
        out_specs=_row_spec(128),
        out_shape=jax.ShapeDtypeStruct((NN, 128), jnp.float32),
    )(x, w)


def _tca_body(deg_ref, p1_ref, dis_ref, h1p_ref):
    d = deg_ref[...]
    dsum = d[0, :, 0:1] + d[1, :, 0:1] + 1.0
    dis = lax.rsqrt(dsum)
    dis_ref[...] = jnp.broadcast_to(dis, (BM, 128))
    h1p_ref[...] = dis * p1_ref[...]


def _tca(deg_p, p1):
    return pl.pallas_call(
        _tca_body,
        grid=(GRID,),
        in_specs=[_part_spec(DEGW), _row_spec(128)],
        out_specs=[_row_spec(128), _row_spec(128)],
        out_shape=[jax.ShapeDtypeStruct((NN, 128), jnp.float32),
                   jax.ShapeDtypeStruct((NN, 128), jnp.float32)],
    )(deg_p, p1)


def _tcb_body(dis_ref, s_ref, hp_ref, b_ref, w_ref, o_ref):
    dis = dis_ref[...]
    s = s_ref[...]
    conv = dis * (s[0] + s[1] + hp_ref[...]) + b_ref[...][None, :]
    h1 = jnp.maximum(conv, 0.0)
    o_ref[...] = dis[:, :64] * _dot(h1, w_ref[...])


def _tcb(dis, s1, h1p, b_e1, w_e2):
    return pl.pallas_call(
        _tcb_body,
        grid=(GRID,),
        in_specs=[_row_spec(128), _part_spec(128), _row_spec(128),
                  _full_spec((128,)), _full_spec((128, 64))],
        out_specs=_row_spec(64),
        out_shape=jax.ShapeDtypeStruct((NN, 64), jnp.float32),
    )(dis, s1, h1p, b_e1, w_e2)


def _tcc_body(dis_ref, s_ref, hp_ref, b2_ref, wfc_ref, bfc_ref, o_ref):
    dis = dis_ref[...][:, :64]
    s = s_ref[...]
    conv = dis * (s[0] + s[1] + hp_ref[...]) + b2_ref[...][None, :]
    z = _dot(conv, wfc_ref[...]) + bfc_ref[...][None, :]
    o_ref[...] = dis * z


def _tcc(dis, s2, h2p, b_e2, w_efc, b_efc):
    return pl.pallas_call(
        _tcc_body,
        grid=(GRID,),
        in_specs=[_row_spec(128), _part_spec(64), _row_spec(64),
                  _full_spec((64,)), _full_spec((64, 64)), _full_spec((64,))],
        out_specs=_row_spec(64),
        out_shape=jax.ShapeDtypeStruct((NN, 64), jnp.float32),
    )(dis, s2, h2p, b_e2, w_efc, b_efc)


def _tcd_body(dis_ref, s_ref, zp_ref, w1_ref, b1_ref, w2_ref, o_ref):
    dis = dis_ref[...]
    dis64 = dis[:, :64]
    s = s_ref[...]
    pz = dis64 * (s[0] + s[1] + zp_ref[...])
    h3 = jnp.maximum(_dot(pz, w1_ref[...]) + b1_ref[...][None, :], 0.0)
    o_ref[...] = dis * _dot(h3, w2_ref[...])


def _tcd(dis, s3, zp, w_d1, b_d1, w_d2):
    return pl.pallas_call(
        _tcd_body,
        grid=(GRID,),
        in_specs=[_row_spec(128), _part_spec(64), _row_spec(64),
                  _full_spec((64, 256)), _full_spec((256,)),
                  _full_spec((256, 128))],
        out_specs=_row_spec(128),
        out_shape=jax.ShapeDtypeStruct((NN, 128), jnp.float32),
    )(dis, s3, zp, w_d1, b_d1, w_d2)


def _tce_body(dis_ref, s_ref, gp_ref, b2_ref, wfc_ref, bfc_ref, o_ref):
    dis = dis_ref[...]
    s = s_ref[...]
    h4 = dis * (s[0] + s[1] + gp_ref[...]) + b2_ref[...][None, :]
    o_ref[...] = _dot(h4, wfc_ref[...]) + bfc_ref[...][None, :]


def _tce(dis, s4, gp, b_d2, w_dfc, b_dfc):
    return pl.pallas_call(
        _tce_body,
        grid=(GRID,),
        in_specs=[_row_spec(128), _part_spec(128), _row_spec(128),
                  _full_spec((128,)), _full_spec((128, 1024)),
                  _full_spec((1024,))],
        out_specs=_row_spec(1024),
        out_shape=jax.ShapeDtypeStruct((NN, 1024), jnp.float32),
    )(dis, s4, gp, b_d2, w_dfc, b_dfc)


def kernel(x, edge_index, w_e1, b_e1, w_e2, b_e2, w_efc, b_efc,
           w_d1, b_d1, w_d2, b_d2, w_dfc, b_dfc):
    npad = EPAD - EE
    src = jnp.concatenate([edge_index[0], jnp.zeros((npad,), jnp.int32)])
    dst = jnp.concatenate(
        [edge_index[1], NN + (jnp.arange(npad, dtype=jnp.int32) % 8)])

    deg_p = _sc_degrees(dst)
    p1 = _tc0(x, w_e1)
    dis, h1p = _tca(deg_p, p1)

    s1 = _prop128(h1p, src, dst)
    h2p = _tcb(dis, s1, h1p, b_e1, w_e2)

    s2 = _prop64(h2p, src, dst)
    zp = _tcc(dis, s2, h2p, b_e2, w_efc, b_efc)

    s3 = _prop64(zp, src, dst)
    gp = _tcd(dis, s3, zp, w_d1, b_d1, w_d2)

    s4 = _prop128(gp, src, dst)
    return _tce(dis, s4, gp, b_d2, w_dfc, b_dfc)


# speedup vs baseline: 10.5542x; 1.0408x over previous
import functools

import jax
import jax.numpy as jnp
from jax import lax
from jax.experimental import pallas as pl
from jax.experimental.pallas import tpu as pltpu
from jax.experimental.pallas import tpu_sc as plsc

NN = 10000
EE = 320000
NC, NS, LANES = 2, 16, 16
NW = NC * NS
B = 128
CH_PER = -(-EE // (B * NW))
EPAD = CH_PER * B * NW
NA = 10240
SLAB = NA // NS
DEGW = 16
BM = 1000
GRID = NN // BM

_mesh = plsc.VectorSubcoreMesh(core_axis_name="c", subcore_axis_name="s")


PH0 = CH_PER // 2 + 1
PH1 = CH_PER - PH0
IDXROWS = PH0 + 1


def _make_prop(F):
    @functools.partial(
        pl.kernel,
        out_type=jax.ShapeDtypeStruct((NC, NA, F), jnp.float32),
        mesh=_mesh,
        scratch_types=[
            pltpu.VMEM((PH0, B), jnp.int32),
            pltpu.VMEM((IDXROWS, B), jnp.int32),
            pltpu.VMEM((B, F), jnp.float32),
            pltpu.VMEM((B, F), jnp.float32),
            pltpu.VMEM_SHARED((NA, F), jnp.float32),
            pltpu.SemaphoreType.DMA,
            pltpu.SemaphoreType.DMA,
        ],
        compiler_params=pltpu.CompilerParams(use_tc_tiling_on_sc=False),
    )
    def prop(h_hbm, src_hbm, dst_hbm, out_hbm, dsti_v, srci_v, rows0_v, rows1_v,
             acc_sh, sem0, sem1):
        cid = lax.axis_index("c")
        sid = lax.axis_index("s")
        wid = sid * NC + cid
        base = wid * CH_PER

        zero16 = jnp.zeros((LANES,), jnp.float32)
        izero16 = jnp.zeros((LANES,), jnp.int32)

        def zb(r, carry):
            for f in range(F // LANES):
                rows0_v[r, pl.ds(f * LANES, LANES)] = zero16
            return carry

        lax.fori_loop(0, B, zb, 0)
        for t in range(SLAB // B):
            pltpu.sync_copy(rows0_v, acc_sh.at[pl.ds(sid * SLAB + t * B, B)])
        for f in range(B // LANES):
            srci_v[PH0, pl.ds(f * LANES, LANES)] = izero16
        plsc.subcore_barrier()

        def gather(j, rows_v, sem):
            pltpu.async_copy(h_hbm.at[srci_v.at[j]], rows_v, sem)

        def gwait(rows_v, sem):
            pltpu.make_async_copy(h_hbm.at[srci_v.at[0]], rows_v, sem).wait()

        def scatter(j, rows_v):
            pltpu.sync_copy(rows_v, acc_sh.at[dsti_v.at[j]], add=True)

        def run_phase(pbase, nch):
            pltpu.sync_copy(src_hbm.at[pl.ds(pbase, nch)],
                            srci_v.at[pl.ds(0, nch)])
            pltpu.sync_copy(dst_hbm.at[pl.ds(pbase, nch)],
                            dsti_v.at[pl.ds(0, nch)])
            gather(0, rows0_v, sem0)

            def step(i, carry):
                a = 2 * i
                gather(a + 1, rows1_v, sem1)
                gwait(rows0_v, sem0)
                scatter(a, rows0_v)
                gather(a + 2, rows0_v, sem0)
                gwait(rows1_v, sem1)
                scatter(a + 1, rows1_v)
                return carry

            lax.fori_loop(0, nch // 2, step, 0)
            gwait(rows0_v, sem0)
            if nch % 2 == 1:
                scatter(nch - 1, rows0_v)

        run_phase(base, PH0)
        run_phase(base + PH0, PH1)
        plsc.subcore_barrier()

        for t in range(SLAB // B):
            row0 = sid * SLAB + t * B
            pltpu.sync_copy(acc_sh.at[pl.ds(row0, B)], rows0_v)
            pltpu.sync_copy(rows0_v, out_hbm.at[cid, pl.ds(row0, B)])

    return prop


_prop128 = _make_prop(128)
_prop64 = _make_prop(64)


@functools.partial(
    pl.kernel,
    out_type=jax.ShapeDtypeStruct((NC, NA, DEGW), jnp.float32),
    mesh=_mesh,
    scratch_types=[
        pltpu.VMEM((CH_PER, B), jnp.int32),
        pltpu.VMEM((B, DEGW), jnp.float32),
        pltpu.VMEM((SLAB, DEGW), jnp.float32),
        pltpu.VMEM_SHARED((NA, DEGW), jnp.float32),
    ],
    compiler_params=pltpu.CompilerParams(use_tc_tiling_on_sc=False),
)
def _sc_degrees(dst_hbm, out_hbm, dsti_v, ones_v, zb_v, acc_sh):
    cid = lax.axis_index("c")
    sid = lax.axis_index("s")
    wid = sid * NC + cid

    one16 = jnp.ones((LANES,), jnp.float32)
    zero16 = jnp.zeros((LANES,), jnp.float32)

    def fill(r, carry):
        ones_v[r] = one16
        return carry

    lax.fori_loop(0, B, fill, 0)

    def zb(r, carry):
        zb_v[r] = zero16
        return carry

    lax.fori_loop(0, SLAB, zb, 0)
    pltpu.sync_copy(zb_v, acc_sh.at[pl.ds(sid * SLAB, SLAB)])
    pltpu.sync_copy(dst_hbm.at[pl.ds(wid * CH_PER, CH_PER)], dsti_v)
    plsc.subcore_barrier()

    def chunk(j, carry):
        pltpu.sync_copy(ones_v, acc_sh.at[dsti_v.at[j]], add=True)
        return carry

    lax.fori_loop(0, CH_PER, chunk, 0)
    plsc.subcore_barrier()

    row0 = sid * SLAB
    pltpu.sync_copy(acc_sh.at[pl.ds(row0, SLAB)], zb_v)
    pltpu.sync_copy(zb_v, out_hbm.at[cid, pl.ds(row0, SLAB)])


def _dot(a, b):
    return jnp.dot(a, b, preferred_element_type=jnp.float32,
                   precision=lax.Precision.HIGHEST)


def _row_spec(f):
    return pl.BlockSpec((BM, f), lambda i: (i, 0))


def _full_spec(shape):
    nd = len(shape)
    return pl.BlockSpec(shape, lambda i, _n=nd: (0,) * _n)


def _part_spec(f):
    return pl.BlockSpec((NC, BM, f), lambda i: (0, i, 0))


def _tc0_body(x_ref, w_ref, o_ref):
    o_ref[...] = _dot(x_ref[...], w_ref[...])


def _tc0(x, w):
    return pl.pallas_call(
        _tc0_body,
        grid=(GRID,),
        in_specs=[_row_spec(128), _full_spec((128, 128))],
        out_specs=_row_spec(128),
        out_shape=jax.ShapeDtypeStruct((NN, 128), jnp.float32),
    )(x, w)


def _tca_body(deg_ref, p1_ref, dis_ref, h1p_ref):
    d = deg_ref[...]
    dsum = d[0, :, 0:1] + d[1, :, 0:1] + 1.0
    dis = lax.rsqrt(dsum)
    dis_ref[...] = jnp.broadcast_to(dis, (BM, 128))
    h1p_ref[...] = dis * p1_ref[...]


def _tca(deg_p, p1):
    return pl.pallas_call(
        _tca_body,
        grid=(GRID,),
        in_specs=[_part_spec(DEGW), _row_spec(128)],
        out_specs=[_row_spec(128), _row_spec(128)],
        out_shape=[jax.ShapeDtypeStruct((NN, 128), jnp.float32),
                   jax.ShapeDtypeStruct((NN, 128), jnp.float32)],
    )(deg_p, p1)


def _tcb_body(dis_ref, s_ref, hp_ref, b_ref, w_ref, o_ref):
    dis = dis_ref[...]
    s = s_ref[...]
    conv = dis * (s[0] + s[1] + hp_ref[...]) + b_ref[...][None, :]
    h1 = jnp.maximum(conv, 0.0)
    o_ref[...] = dis[:, :64] * _dot(h1, w_ref[...])


def _tcb(dis, s1, h1p, b_e1, w_e2):
    return pl.pallas_call(
        _tcb_body,
        grid=(GRID,),
        in_specs=[_row_spec(128), _part_spec(128), _row_spec(128),
                  _full_spec((128,)), _full_spec((128, 64))],
        out_specs=_row_spec(64),
        out_shape=jax.ShapeDtypeStruct((NN, 64), jnp.float32),
    )(dis, s1, h1p, b_e1, w_e2)


def _tcc_body(dis_ref, s_ref, hp_ref, b2_ref, wfc_ref, bfc_ref, o_ref):
    dis = dis_ref[...][:, :64]
    s = s_ref[...]
    conv = dis * (s[0] + s[1] + hp_ref[...]) + b2_ref[...][None, :]
    z = _dot(conv, wfc_ref[...]) + bfc_ref[...][None, :]
    o_ref[...] = dis * z


def _tcc(dis, s2, h2p, b_e2, w_efc, b_efc):
    return pl.pallas_call(
        _tcc_body,
        grid=(GRID,),
        in_specs=[_row_spec(128), _part_spec(64), _row_spec(64),
                  _full_spec((64,)), _full_spec((64, 64)), _full_spec((64,))],
        out_specs=_row_spec(64),
        out_shape=jax.ShapeDtypeStruct((NN, 64), jnp.float32),
    )(dis, s2, h2p, b_e2, w_efc, b_efc)


def _tcd_body(dis_ref, s_ref, zp_ref, w1_ref, b1_ref, w2_ref, o_ref):
    dis = dis_ref[...]
    dis64 = dis[:, :64]
    s = s_ref[...]
    pz = dis64 * (s[0] + s[1] + zp_ref[...])
    h3 = jnp.maximum(_dot(pz, w1_ref[...]) + b1_ref[...][None, :], 0.0)
    o_ref[...] = dis * _dot(h3, w2_ref[...])


def _tcd(dis, s3, zp, w_d1, b_d1, w_d2):
    return pl.pallas_call(
        _tcd_body,
        grid=(GRID,),
        in_specs=[_row_spec(128), _part_spec(64), _row_spec(64),
                  _full_spec((64, 256)), _full_spec((256,)),
                  _full_spec((256, 128))],
        out_specs=_row_spec(128),
        out_shape=jax.ShapeDtypeStruct((NN, 128), jnp.float32),
    )(dis, s3, zp, w_d1, b_d1, w_d2)


def _tce_body(dis_ref, s_ref, gp_ref, b2_ref, wfc_ref, bfc_ref, o_ref):
    dis = dis_ref[...]
    s = s_ref[...]
    h4 = dis * (s[0] + s[1] + gp_ref[...]) + b2_ref[...][None, :]
    o_ref[...] = _dot(h4, wfc_ref[...]) + bfc_ref[...][None, :]


def _tce(dis, s4, gp, b_d2, w_dfc, b_dfc):
    return pl.pallas_call(
        _tce_body,
        grid=(GRID,),
        in_specs=[_row_spec(128), _part_spec(128), _row_spec(128),
                  _full_spec((128,)), _full_spec((128, 1024)),
                  _full_spec((1024,))],
        out_specs=_row_spec(1024),
        out_shape=jax.ShapeDtypeStruct((NN, 1024), jnp.float32),
    )(dis, s4, gp, b_d2, w_dfc, b_dfc)


def kernel(x, edge_index, w_e1, b_e1, w_e2, b_e2, w_efc, b_efc,
           w_d1, b_d1, w_d2, b_d2, w_dfc, b_dfc):
    npad = EPAD - EE
    src = jnp.concatenate(
        [edge_index[0], jnp.zeros((npad,), jnp.int32)]).reshape(-1, B)
    dst = jnp.concatenate(
        [edge_index[1],
         NN + (jnp.arange(npad, dtype=jnp.int32) % 8)]).reshape(-1, B)

    deg_p = _sc_degrees(dst)
    p1 = _tc0(x, w_e1)
    dis, h1p = _tca(deg_p, p1)

    s1 = _prop128(h1p, src, dst)
    h2p = _tcb(dis, s1, h1p, b_e1, w_e2)

    s2 = _prop64(h2p, src, dst)
    zp = _tcc(dis, s2, h2p, b_e2, w_efc, b_efc)

    s3 = _prop64(zp, src, dst)
    gp = _tcd(dis, s3, zp, w_d1, b_d1, w_d2)

    s4 = _prop128(gp, src, dst)
    return _tce(dis, s4, gp, b_d2, w_dfc, b_dfc)


# speedup vs baseline: 12.9488x; 1.2269x over previous
import functools

import jax
import jax.numpy as jnp
from jax import lax
from jax.experimental import pallas as pl
from jax.experimental.pallas import tpu as pltpu
from jax.experimental.pallas import tpu_sc as plsc

NN = 10000
EE = 320000
NC, NS, LANES = 2, 16, 16
NW = NC * NS
B = 128
CH_PER = -(-EE // (B * NW))
EPAD = CH_PER * B * NW
NA = 10240
SLAB = NA // NS
DEGW = 16
BM = 1000
GRID = NN // BM

_mesh = plsc.VectorSubcoreMesh(core_axis_name="c", subcore_axis_name="s")


PH0 = CH_PER // 2 + 1
PH1 = CH_PER - PH0
IDXROWS = PH0 + 1


def _make_prop(F, stage_h=False):
    scratch = [
        pltpu.VMEM((PH0, B), jnp.int32),
        pltpu.VMEM((IDXROWS, B), jnp.int32),
        pltpu.VMEM((B, F), jnp.float32),
        pltpu.VMEM((B, F), jnp.float32),
        pltpu.VMEM_SHARED((NA, F), jnp.float32),
        pltpu.SemaphoreType.DMA,
        pltpu.SemaphoreType.DMA,
    ]
    if stage_h:
        scratch.append(pltpu.VMEM_SHARED((NN, F), jnp.float32))

    @functools.partial(
        pl.kernel,
        out_type=jax.ShapeDtypeStruct((NC, NA, F), jnp.float32),
        mesh=_mesh,
        scratch_types=scratch,
        compiler_params=pltpu.CompilerParams(use_tc_tiling_on_sc=False),
    )
    def prop(h_hbm, src_hbm, dst_hbm, out_hbm, dsti_v, srci_v, rows0_v, rows1_v,
             acc_sh, sem0, sem1, *maybe_h_sh):
        cid = lax.axis_index("c")
        sid = lax.axis_index("s")
        wid = sid * NC + cid
        base = wid * CH_PER
        if stage_h:
            h_src = maybe_h_sh[0]
            hrows = NN // NS
            pltpu.sync_copy(h_hbm.at[pl.ds(sid * hrows, hrows)],
                            h_src.at[pl.ds(sid * hrows, hrows)])
        else:
            h_src = h_hbm

        zero16 = jnp.zeros((LANES,), jnp.float32)
        izero16 = jnp.zeros((LANES,), jnp.int32)

        def zb(r, carry):
            for f in range(F // LANES):
                rows0_v[r, pl.ds(f * LANES, LANES)] = zero16
            return carry

        lax.fori_loop(0, B, zb, 0)
        for t in range(SLAB // B):
            pltpu.sync_copy(rows0_v, acc_sh.at[pl.ds(sid * SLAB + t * B, B)])
        for f in range(B // LANES):
            srci_v[PH0, pl.ds(f * LANES, LANES)] = izero16
        plsc.subcore_barrier()

        def gather(j, rows_v, sem):
            pltpu.async_copy(h_src.at[srci_v.at[j]], rows_v, sem)

        def gwait(rows_v, sem):
            pltpu.make_async_copy(h_src.at[srci_v.at[0]], rows_v, sem).wait()

        def scatter(j, rows_v):
            pltpu.sync_copy(rows_v, acc_sh.at[dsti_v.at[j]], add=True)

        def run_phase(pbase, nch):
            pltpu.sync_copy(src_hbm.at[pl.ds(pbase, nch)],
                            srci_v.at[pl.ds(0, nch)])
            pltpu.sync_copy(dst_hbm.at[pl.ds(pbase, nch)],
                            dsti_v.at[pl.ds(0, nch)])
            gather(0, rows0_v, sem0)

            def step(i, carry):
                a = 2 * i
                gather(a + 1, rows1_v, sem1)
                gwait(rows0_v, sem0)
                scatter(a, rows0_v)
                gather(a + 2, rows0_v, sem0)
                gwait(rows1_v, sem1)
                scatter(a + 1, rows1_v)
                return carry

            lax.fori_loop(0, nch // 2, step, 0)
            gwait(rows0_v, sem0)
            if nch % 2 == 1:
                scatter(nch - 1, rows0_v)

        run_phase(base, PH0)
        run_phase(base + PH0, PH1)
        plsc.subcore_barrier()

        for t in range(SLAB // B):
            row0 = sid * SLAB + t * B
            pltpu.sync_copy(acc_sh.at[pl.ds(row0, B)], rows0_v)
            pltpu.sync_copy(rows0_v, out_hbm.at[cid, pl.ds(row0, B)])

    return prop


_prop128 = _make_prop(128)
_prop64 = _make_prop(64, stage_h=True)


@functools.partial(
    pl.kernel,
    out_type=jax.ShapeDtypeStruct((NC, NA, DEGW), jnp.float32),
    mesh=_mesh,
    scratch_types=[
        pltpu.VMEM((CH_PER, B), jnp.int32),
        pltpu.VMEM((B, DEGW), jnp.float32),
        pltpu.VMEM((SLAB, DEGW), jnp.float32),
        pltpu.VMEM_SHARED((NA, DEGW), jnp.float32),
    ],
    compiler_params=pltpu.CompilerParams(use_tc_tiling_on_sc=False),
)
def _sc_degrees(dst_hbm, out_hbm, dsti_v, ones_v, zb_v, acc_sh):
    cid = lax.axis_index("c")
    sid = lax.axis_index("s")
    wid = sid * NC + cid

    one16 = jnp.ones((LANES,), jnp.float32)
    zero16 = jnp.zeros((LANES,), jnp.float32)

    def fill(r, carry):
        ones_v[r] = one16
        return carry

    lax.fori_loop(0, B, fill, 0)

    def zb(r, carry):
        zb_v[r] = zero16
        return carry

    lax.fori_loop(0, SLAB, zb, 0)
    pltpu.sync_copy(zb_v, acc_sh.at[pl.ds(sid * SLAB, SLAB)])
    pltpu.sync_copy(dst_hbm.at[pl.ds(wid * CH_PER, CH_PER)], dsti_v)
    plsc.subcore_barrier()

    def chunk(j, carry):
        pltpu.sync_copy(ones_v, acc_sh.at[dsti_v.at[j]], add=True)
        return carry

    lax.fori_loop(0, CH_PER, chunk, 0)
    plsc.subcore_barrier()

    row0 = sid * SLAB
    pltpu.sync_copy(acc_sh.at[pl.ds(row0, SLAB)], zb_v)
    pltpu.sync_copy(zb_v, out_hbm.at[cid, pl.ds(row0, SLAB)])


def _dot(a, b):
    return jnp.dot(a, b, preferred_element_type=jnp.float32,
                   precision=lax.Precision.HIGHEST)


def _row_spec(f):
    return pl.BlockSpec((BM, f), lambda i: (i, 0))


def _full_spec(shape):
    nd = len(shape)
    return pl.BlockSpec(shape, lambda i, _n=nd: (0,) * _n)


def _part_spec(f):
    return pl.BlockSpec((NC, BM, f), lambda i: (0, i, 0))


def _tc0_body(x_ref, w_ref, o_ref):
    o_ref[...] = _dot(x_ref[...], w_ref[...])


def _tc0(x, w):
    return pl.pallas_call(
        _tc0_body,
        grid=(GRID,),
        in_specs=[_row_spec(128), _full_spec((128, 128))],
        out_specs=_row_spec(128),
        out_shape=jax.ShapeDtypeStruct((NN, 128), jnp.float32),
    )(x, w)


def _tca_body(deg_ref, p1_ref, dis_ref, h1p_ref):
    d = deg_ref[...]
    dsum = d[0, :, 0:1] + d[1, :, 0:1] + 1.0
    dis = lax.rsqrt(dsum)
    dis_ref[...] = jnp.broadcast_to(dis, (BM, 128))
    h1p_ref[...] = dis * p1_ref[...]


def _tca(deg_p, p1):
    return pl.pallas_call(
        _tca_body,
        grid=(GRID,),
        in_specs=[_part_spec(DEGW), _row_spec(128)],
        out_specs=[_row_spec(128), _row_spec(128)],
        out_shape=[jax.ShapeDtypeStruct((NN, 128), jnp.float32),
                   jax.ShapeDtypeStruct((NN, 128), jnp.float32)],
    )(deg_p, p1)


def _tcb_body(dis_ref, s_ref, hp_ref, b_ref, w_ref, o_ref):
    dis = dis_ref[...]
    s = s_ref[...]
    conv = dis * (s[0] + s[1] + hp_ref[...]) + b_ref[...][None, :]
    h1 = jnp.maximum(conv, 0.0)
    o_ref[...] = dis[:, :64] * _dot(h1, w_ref[...])


def _tcb(dis, s1, h1p, b_e1, w_e2):
    return pl.pallas_call(
        _tcb_body,
        grid=(GRID,),
        in_specs=[_row_spec(128), _part_spec(128), _row_spec(128),
                  _full_spec((128,)), _full_spec((128, 64))],
        out_specs=_row_spec(64),
        out_shape=jax.ShapeDtypeStruct((NN, 64), jnp.float32),
    )(dis, s1, h1p, b_e1, w_e2)


def _tcc_body(dis_ref, s_ref, hp_ref, b2_ref, wfc_ref, bfc_ref, o_ref):
    dis = dis_ref[...][:, :64]
    s = s_ref[...]
    conv = dis * (s[0] + s[1] + hp_ref[...]) + b2_ref[...][None, :]
    z = _dot(conv, wfc_ref[...]) + bfc_ref[...][None, :]
    o_ref[...] = dis * z


def _tcc(dis, s2, h2p, b_e2, w_efc, b_efc):
    return pl.pallas_call(
        _tcc_body,
        grid=(GRID,),
        in_specs=[_row_spec(128), _part_spec(64), _row_spec(64),
                  _full_spec((64,)), _full_spec((64, 64)), _full_spec((64,))],
        out_specs=_row_spec(64),
        out_shape=jax.ShapeDtypeStruct((NN, 64), jnp.float32),
    )(dis, s2, h2p, b_e2, w_efc, b_efc)


def _tcd_body(dis_ref, s_ref, zp_ref, w1_ref, b1_ref, w2_ref, o_ref):
    dis = dis_ref[...]
    dis64 = dis[:, :64]
    s = s_ref[...]
    pz = dis64 * (s[0] + s[1] + zp_ref[...])
    h3 = jnp.maximum(_dot(pz, w1_ref[...]) + b1_ref[...][None, :], 0.0)
    o_ref[...] = dis * _dot(h3, w2_ref[...])


def _tcd(dis, s3, zp, w_d1, b_d1, w_d2):
    return pl.pallas_call(
        _tcd_body,
        grid=(GRID,),
        in_specs=[_row_spec(128), _part_spec(64), _row_spec(64),
                  _full_spec((64, 256)), _full_spec((256,)),
                  _full_spec((256, 128))],
        out_specs=_row_spec(128),
        out_shape=jax.ShapeDtypeStruct((NN, 128), jnp.float32),
    )(dis, s3, zp, w_d1, b_d1, w_d2)


def _tce_body(dis_ref, s_ref, gp_ref, b2_ref, wfc_ref, bfc_ref, o_ref):
    dis = dis_ref[...]
    s = s_ref[...]
    h4 = dis * (s[0] + s[1] + gp_ref[...]) + b2_ref[...][None, :]
    o_ref[...] = _dot(h4, wfc_ref[...]) + bfc_ref[...][None, :]


def _tce(dis, s4, gp, b_d2, w_dfc, b_dfc):
    return pl.pallas_call(
        _tce_body,
        grid=(GRID,),
        in_specs=[_row_spec(128), _part_spec(128), _row_spec(128),
                  _full_spec((128,)), _full_spec((128, 1024)),
                  _full_spec((1024,))],
        out_specs=_row_spec(1024),
        out_shape=jax.ShapeDtypeStruct((NN, 1024), jnp.float32),
    )(dis, s4, gp, b_d2, w_dfc, b_dfc)


def kernel(x, edge_index, w_e1, b_e1, w_e2, b_e2, w_efc, b_efc,
           w_d1, b_d1, w_d2, b_d2, w_dfc, b_dfc):
    npad = EPAD - EE
    src = jnp.concatenate(
        [edge_index[0], jnp.zeros((npad,), jnp.int32)]).reshape(-1, B)
    dst = jnp.concatenate(
        [edge_index[1],
         NN + (jnp.arange(npad, dtype=jnp.int32) % 8)]).reshape(-1, B)

    deg_p = _sc_degrees(dst)
    p1 = _tc0(x, w_e1)
    dis, h1p = _tca(deg_p, p1)

    s1 = _prop128(h1p, src, dst)
    h2p = _tcb(dis, s1, h1p, b_e1, w_e2)

    s2 = _prop64(h2p, src, dst)
    zp = _tcc(dis, s2, h2p, b_e2, w_efc, b_efc)

    s3 = _prop64(zp, src, dst)
    gp = _tcd(dis, s3, zp, w_d1, b_d1, w_d2)

    s4 = _prop128(gp, src, dst)
    return _tce(dis, s4, gp, b_d2, w_dfc, b_dfc)


# speedup vs baseline: 21.6585x; 1.6726x over previous
import functools

import jax
import jax.numpy as jnp
from jax import lax
from jax.experimental import pallas as pl
from jax.experimental.pallas import tpu as pltpu
from jax.experimental.pallas import tpu_sc as plsc

NN = 10000
EE = 320000
NC, NS, LANES = 2, 16, 16
NW = NC * NS
B = 128
CH_PER = -(-EE // (B * NW))
EPAD = CH_PER * B * NW
NA = 10240
SLAB = NA // NS
DEGW = 16
BM = 1000
GRID = NN // BM

_mesh = plsc.VectorSubcoreMesh(core_axis_name="c", subcore_axis_name="s")


PH0 = CH_PER // 2 + 1
PH1 = CH_PER - PH0
IDXROWS = PH0 + 1


def _make_prop(F, stage_h=False):
    scratch = [
        pltpu.VMEM((PH0, B), jnp.int32),
        pltpu.VMEM((IDXROWS, B), jnp.int32),
        pltpu.VMEM((B, F), jnp.float32),
        pltpu.VMEM((B, F), jnp.float32),
        pltpu.VMEM_SHARED((NA, F), jnp.float32),
        pltpu.SemaphoreType.DMA,
        pltpu.SemaphoreType.DMA,
    ]
    if stage_h:
        scratch.append(pltpu.VMEM_SHARED((NN, F), jnp.float32))

    @functools.partial(
        pl.kernel,
        out_type=jax.ShapeDtypeStruct((NC, NA, F), jnp.float32),
        mesh=_mesh,
        scratch_types=scratch,
        compiler_params=pltpu.CompilerParams(use_tc_tiling_on_sc=False),
    )
    def prop(h_hbm, src_hbm, dst_hbm, out_hbm, dsti_v, srci_v, rows0_v, rows1_v,
             acc_sh, sem0, sem1, *maybe_h_sh):
        cid = lax.axis_index("c")
        sid = lax.axis_index("s")
        wid = sid * NC + cid
        base = wid * CH_PER
        if stage_h:
            h_src = maybe_h_sh[0]
            hrows = NN // NS
            pltpu.sync_copy(h_hbm.at[pl.ds(sid * hrows, hrows)],
                            h_src.at[pl.ds(sid * hrows, hrows)])
        else:
            h_src = h_hbm

        zero16 = jnp.zeros((LANES,), jnp.float32)
        izero16 = jnp.zeros((LANES,), jnp.int32)

        def zb(r, carry):
            for f in range(F // LANES):
                rows0_v[r, pl.ds(f * LANES, LANES)] = zero16
            return carry

        lax.fori_loop(0, B, zb, 0)
        for t in range(SLAB // B):
            pltpu.sync_copy(rows0_v, acc_sh.at[pl.ds(sid * SLAB + t * B, B)])
        for f in range(B // LANES):
            srci_v[PH0, pl.ds(f * LANES, LANES)] = izero16
        plsc.subcore_barrier()

        def gather(j, rows_v, sem):
            pltpu.async_copy(h_src.at[srci_v.at[j]], rows_v, sem)

        def gwait(rows_v, sem):
            pltpu.make_async_copy(h_src.at[srci_v.at[0]], rows_v, sem).wait()

        def scatter(j, rows_v):
            pltpu.sync_copy(rows_v, acc_sh.at[dsti_v.at[j]], add=True)

        def run_phase(pbase, nch):
            pltpu.sync_copy(src_hbm.at[pl.ds(pbase, nch)],
                            srci_v.at[pl.ds(0, nch)])
            pltpu.sync_copy(dst_hbm.at[pl.ds(pbase, nch)],
                            dsti_v.at[pl.ds(0, nch)])
            gather(0, rows0_v, sem0)

            def step(i, carry):
                a = 2 * i
                gather(a + 1, rows1_v, sem1)
                gwait(rows0_v, sem0)
                scatter(a, rows0_v)
                gather(a + 2, rows0_v, sem0)
                gwait(rows1_v, sem1)
                scatter(a + 1, rows1_v)
                return carry

            lax.fori_loop(0, nch // 2, step, 0)
            gwait(rows0_v, sem0)
            if nch % 2 == 1:
                scatter(nch - 1, rows0_v)

        run_phase(base, PH0)
        run_phase(base + PH0, PH1)
        plsc.subcore_barrier()

        for t in range(SLAB // B):
            row0 = sid * SLAB + t * B
            pltpu.sync_copy(acc_sh.at[pl.ds(row0, B)], rows0_v)
            pltpu.sync_copy(rows0_v, out_hbm.at[cid, pl.ds(row0, B)])

    return prop


_prop64 = _make_prop(64, stage_h=True)

FH = 64
CH_TILE = NCHUNK_ALL = EPAD // B // NS
_PHASES = (40, 40, 40, 38)


@functools.partial(
    pl.kernel,
    out_type=jax.ShapeDtypeStruct((NC, NA, FH), jnp.float32),
    mesh=_mesh,
    scratch_types=[
        pltpu.VMEM((PH0, B), jnp.int32),
        pltpu.VMEM((IDXROWS, B), jnp.int32),
        pltpu.VMEM((B, FH), jnp.float32),
        pltpu.VMEM((B, FH), jnp.float32),
        pltpu.VMEM_SHARED((NA, FH), jnp.float32),
        pltpu.VMEM_SHARED((NN, FH), jnp.float32),
        pltpu.SemaphoreType.DMA,
        pltpu.SemaphoreType.DMA,
    ],
    compiler_params=pltpu.CompilerParams(use_tc_tiling_on_sc=False),
)
def _prop128(h3_hbm, src_hbm, dst_hbm, out_hbm, dsti_v, srci_v, rows0_v,
             rows1_v, acc_sh, h_sh, sem0, sem1):
    cid = lax.axis_index("c")
    sid = lax.axis_index("s")
    base = sid * CH_TILE

    zero16 = jnp.zeros((LANES,), jnp.float32)
    izero16 = jnp.zeros((LANES,), jnp.int32)

    hrows = NN // NS
    pltpu.sync_copy(h3_hbm.at[cid, pl.ds(sid * hrows, hrows)],
                    h_sh.at[pl.ds(sid * hrows, hrows)])

    def zb(r, carry):
        for f in range(FH // LANES):
            rows0_v[r, pl.ds(f * LANES, LANES)] = zero16
        return carry

    lax.fori_loop(0, B, zb, 0)
    for t in range(SLAB // B):
        pltpu.sync_copy(rows0_v, acc_sh.at[pl.ds(sid * SLAB + t * B, B)])
    for f in range(B // LANES):
        srci_v[PH0, pl.ds(f * LANES, LANES)] = izero16
    plsc.subcore_barrier()

    def gather(j, rows_v, sem):
        pltpu.async_copy(h_sh.at[srci_v.at[j]], rows_v, sem)

    def gwait(rows_v, sem):
        pltpu.make_async_copy(h_sh.at[srci_v.at[0]], rows_v, sem).wait()

    def scatter(j, rows_v):
        pltpu.sync_copy(rows_v, acc_sh.at[dsti_v.at[j]], add=True)

    def run_phase(pbase, nch):
        pltpu.sync_copy(src_hbm.at[pl.ds(pbase, nch)], srci_v.at[pl.ds(0, nch)])
        pltpu.sync_copy(dst_hbm.at[pl.ds(pbase, nch)], dsti_v.at[pl.ds(0, nch)])
        gather(0, rows0_v, sem0)

        def step(i, carry):
            a = 2 * i
            gather(a + 1, rows1_v, sem1)
            gwait(rows0_v, sem0)
            scatter(a, rows0_v)
            gather(a + 2, rows0_v, sem0)
            gwait(rows1_v, sem1)
            scatter(a + 1, rows1_v)
            return carry

        lax.fori_loop(0, nch // 2, step, 0)
        gwait(rows0_v, sem0)
        if nch % 2 == 1:
            scatter(nch - 1, rows0_v)

    off = 0
    for nch in _PHASES:
        run_phase(base + off, nch)
        off += nch
    plsc.subcore_barrier()

    for t in range(SLAB // B):
        row0 = sid * SLAB + t * B
        pltpu.sync_copy(acc_sh.at[pl.ds(row0, B)], rows0_v)
        pltpu.sync_copy(rows0_v, out_hbm.at[cid, pl.ds(row0, B)])


@functools.partial(
    pl.kernel,
    out_type=jax.ShapeDtypeStruct((NC, NA, DEGW), jnp.float32),
    mesh=_mesh,
    scratch_types=[
        pltpu.VMEM((CH_PER, B), jnp.int32),
        pltpu.VMEM((B, DEGW), jnp.float32),
        pltpu.VMEM((SLAB, DEGW), jnp.float32),
        pltpu.VMEM_SHARED((NA, DEGW), jnp.float32),
    ],
    compiler_params=pltpu.CompilerParams(use_tc_tiling_on_sc=False),
)
def _sc_degrees(dst_hbm, out_hbm, dsti_v, ones_v, zb_v, acc_sh):
    cid = lax.axis_index("c")
    sid = lax.axis_index("s")
    wid = sid * NC + cid

    one16 = jnp.ones((LANES,), jnp.float32)
    zero16 = jnp.zeros((LANES,), jnp.float32)

    def fill(r, carry):
        ones_v[r] = one16
        return carry

    lax.fori_loop(0, B, fill, 0)

    def zb(r, carry):
        zb_v[r] = zero16
        return carry

    lax.fori_loop(0, SLAB, zb, 0)
    pltpu.sync_copy(zb_v, acc_sh.at[pl.ds(sid * SLAB, SLAB)])
    pltpu.sync_copy(dst_hbm.at[pl.ds(wid * CH_PER, CH_PER)], dsti_v)
    plsc.subcore_barrier()

    def chunk(j, carry):
        pltpu.sync_copy(ones_v, acc_sh.at[dsti_v.at[j]], add=True)
        return carry

    lax.fori_loop(0, CH_PER, chunk, 0)
    plsc.subcore_barrier()

    row0 = sid * SLAB
    pltpu.sync_copy(acc_sh.at[pl.ds(row0, SLAB)], zb_v)
    pltpu.sync_copy(zb_v, out_hbm.at[cid, pl.ds(row0, SLAB)])


def _dot(a, b):
    return jnp.dot(a, b, preferred_element_type=jnp.float32,
                   precision=lax.Precision.HIGHEST)


def _row_spec(f):
    return pl.BlockSpec((BM, f), lambda i: (i, 0))


def _full_spec(shape):
    nd = len(shape)
    return pl.BlockSpec(shape, lambda i, _n=nd: (0,) * _n)


def _part_spec(f):
    return pl.BlockSpec((NC, BM, f), lambda i: (0, i, 0))


def _tc0_body(x_ref, w_ref, o_ref):
    o_ref[...] = _dot(x_ref[...], w_ref[...])


def _tc0(x, w):
    return pl.pallas_call(
        _tc0_body,
        grid=(GRID,),
        in_specs=[_row_spec(128), _full_spec((128, 128))],
        out_specs=_row_spec(128),
        out_shape=jax.ShapeDtypeStruct((NN, 128), jnp.float32),
    )(x, w)


def _tca_body(deg_ref, p1_ref, dis_ref, h1p_ref):
    d = deg_ref[...]
    dsum = d[0, :, 0:1] + d[1, :, 0:1] + 1.0
    dis = lax.rsqrt(dsum)
    dis_ref[...] = jnp.broadcast_to(dis, (BM, 128))
    h1p = dis * p1_ref[...]
    h1p_ref[0] = h1p[:, :FH]
    h1p_ref[1] = h1p[:, FH:]


def _tca(deg_p, p1):
    return pl.pallas_call(
        _tca_body,
        grid=(GRID,),
        in_specs=[_part_spec(DEGW), _row_spec(128)],
        out_specs=[_row_spec(128), _part_spec(FH)],
        out_shape=[jax.ShapeDtypeStruct((NN, 128), jnp.float32),
                   jax.ShapeDtypeStruct((NC, NN, FH), jnp.float32)],
    )(deg_p, p1)


def _tcb_body(dis_ref, s_ref, hp_ref, b_ref, w_ref, o_ref):
    dis = dis_ref[...]
    s = s_ref[...]
    hp = hp_ref[...]
    ah = jnp.concatenate([s[0] + hp[0], s[1] + hp[1]], axis=-1)
    conv = dis * ah + b_ref[...][None, :]
    h1 = jnp.maximum(conv, 0.0)
    o_ref[...] = dis[:, :64] * _dot(h1, w_ref[...])


def _tcb(dis, s1, h1p, b_e1, w_e2):
    return pl.pallas_call(
        _tcb_body,
        grid=(GRID,),
        in_specs=[_row_spec(128), _part_spec(FH), _part_spec(FH),
                  _full_spec((128,)), _full_spec((128, 64))],
        out_specs=_row_spec(64),
        out_shape=jax.ShapeDtypeStruct((NN, 64), jnp.float32),
    )(dis, s1, h1p, b_e1, w_e2)


def _tcc_body(dis_ref, s_ref, hp_ref, b2_ref, wfc_ref, bfc_ref, o_ref):
    dis = dis_ref[...][:, :64]
    s = s_ref[...]
    conv = dis * (s[0] + s[1] + hp_ref[...]) + b2_ref[...][None, :]
    z = _dot(conv, wfc_ref[...]) + bfc_ref[...][None, :]
    o_ref[...] = dis * z


def _tcc(dis, s2, h2p, b_e2, w_efc, b_efc):
    return pl.pallas_call(
        _tcc_body,
        grid=(GRID,),
        in_specs=[_row_spec(128), _part_spec(64), _row_spec(64),
                  _full_spec((64,)), _full_spec((64, 64)), _full_spec((64,))],
        out_specs=_row_spec(64),
        out_shape=jax.ShapeDtypeStruct((NN, 64), jnp.float32),
    )(dis, s2, h2p, b_e2, w_efc, b_efc)


def _tcd_body(dis_ref, s_ref, zp_ref, w1_ref, b1_ref, w2_ref, o_ref):
    dis = dis_ref[...]
    dis64 = dis[:, :64]
    s = s_ref[...]
    pz = dis64 * (s[0] + s[1] + zp_ref[...])
    h3 = jnp.maximum(_dot(pz, w1_ref[...]) + b1_ref[...][None, :], 0.0)
    gp = dis * _dot(h3, w2_ref[...])
    o_ref[0] = gp[:, :FH]
    o_ref[1] = gp[:, FH:]


def _tcd(dis, s3, zp, w_d1, b_d1, w_d2):
    return pl.pallas_call(
        _tcd_body,
        grid=(GRID,),
        in_specs=[_row_spec(128), _part_spec(64), _row_spec(64),
                  _full_spec((64, 256)), _full_spec((256,)),
                  _full_spec((256, 128))],
        out_specs=_part_spec(FH),
        out_shape=jax.ShapeDtypeStruct((NC, NN, FH), jnp.float32),
    )(dis, s3, zp, w_d1, b_d1, w_d2)


def _tce_body(dis_ref, s_ref, gp_ref, b2_ref, wfc_ref, bfc_ref, o_ref):
    dis = dis_ref[...]
    s = s_ref[...]
    gp = gp_ref[...]
    ah = jnp.concatenate([s[0] + gp[0], s[1] + gp[1]], axis=-1)
    h4 = dis * ah + b2_ref[...][None, :]
    o_ref[...] = _dot(h4, wfc_ref[...]) + bfc_ref[...][None, :]


def _tce(dis, s4, gp, b_d2, w_dfc, b_dfc):
    return pl.pallas_call(
        _tce_body,
        grid=(GRID,),
        in_specs=[_row_spec(128), _part_spec(FH), _part_spec(FH),
                  _full_spec((128,)), _full_spec((128, 1024)),
                  _full_spec((1024,))],
        out_specs=_row_spec(1024),
        out_shape=jax.ShapeDtypeStruct((NN, 1024), jnp.float32),
    )(dis, s4, gp, b_d2, w_dfc, b_dfc)


def kernel(x, edge_index, w_e1, b_e1, w_e2, b_e2, w_efc, b_efc,
           w_d1, b_d1, w_d2, b_d2, w_dfc, b_dfc):
    npad = EPAD - EE
    src = jnp.concatenate(
        [edge_index[0], jnp.zeros((npad,), jnp.int32)]).reshape(-1, B)
    dst = jnp.concatenate(
        [edge_index[1],
         NN + (jnp.arange(npad, dtype=jnp.int32) % 8)]).reshape(-1, B)

    deg_p = _sc_degrees(dst)
    p1 = _tc0(x, w_e1)
    dis, h1p = _tca(deg_p, p1)

    s1 = _prop128(h1p, src, dst)
    h2p = _tcb(dis, s1, h1p, b_e1, w_e2)

    s2 = _prop64(h2p, src, dst)
    zp = _tcc(dis, s2, h2p, b_e2, w_efc, b_efc)

    s3 = _prop64(zp, src, dst)
    gp = _tcd(dis, s3, zp, w_d1, b_d1, w_d2)

    s4 = _prop128(gp, src, dst)
    return _tce(dis, s4, gp, b_d2, w_dfc, b_dfc)


# speedup vs baseline: 23.0058x; 1.0622x over previous
import functools

import jax
import jax.numpy as jnp
from jax import lax
from jax.experimental import pallas as pl
from jax.experimental.pallas import tpu as pltpu
from jax.experimental.pallas import tpu_sc as plsc

NN = 10000
EE = 320000
NC, NS, LANES = 2, 16, 16
NW = NC * NS
B = 128
CH_PER = -(-EE // (B * NW))
EPAD = CH_PER * B * NW
NA = 10240
SLAB = NA // NS
DEGW = 16
BM = 1000
GRID = NN // BM

_mesh = plsc.VectorSubcoreMesh(core_axis_name="c", subcore_axis_name="s")


PH0 = CH_PER // 2 + 1
PH1 = CH_PER - PH0
IDXROWS = PH0 + 1


def _make_prop(F, stage_h=False):
    scratch = [
        pltpu.VMEM((PH0, B), jnp.int32),
        pltpu.VMEM((IDXROWS, B), jnp.int32),
        pltpu.VMEM((B, F), jnp.float32),
        pltpu.VMEM((B, F), jnp.float32),
        pltpu.VMEM_SHARED((NA, F), jnp.float32),
        pltpu.SemaphoreType.DMA,
        pltpu.SemaphoreType.DMA,
    ]
    if stage_h:
        scratch.append(pltpu.VMEM_SHARED((NN, F), jnp.float32))

    @functools.partial(
        pl.kernel,
        out_type=jax.ShapeDtypeStruct((NC, NA, F), jnp.float32),
        mesh=_mesh,
        scratch_types=scratch,
        compiler_params=pltpu.CompilerParams(use_tc_tiling_on_sc=False),
    )
    def prop(h_hbm, src_hbm, dst_hbm, out_hbm, dsti_v, srci_v, rows0_v, rows1_v,
             acc_sh, sem0, sem1, *maybe_h_sh):
        cid = lax.axis_index("c")
        sid = lax.axis_index("s")
        wid = sid * NC + cid
        base = wid * CH_PER
        if stage_h:
            h_src = maybe_h_sh[0]
            hrows = NN // NS
            pltpu.sync_copy(h_hbm.at[pl.ds(sid * hrows, hrows)],
                            h_src.at[pl.ds(sid * hrows, hrows)])
        else:
            h_src = h_hbm

        zero16 = jnp.zeros((LANES,), jnp.float32)
        izero16 = jnp.zeros((LANES,), jnp.int32)

        def zb(r, carry):
            for f in range(F // LANES):
                rows0_v[r, pl.ds(f * LANES, LANES)] = zero16
            return carry

        lax.fori_loop(0, B, zb, 0)
        for t in range(SLAB // B):
            pltpu.sync_copy(rows0_v, acc_sh.at[pl.ds(sid * SLAB + t * B, B)])
        for f in range(B // LANES):
            srci_v[PH0, pl.ds(f * LANES, LANES)] = izero16
        plsc.subcore_barrier()

        def gather(j, rows_v, sem):
            pltpu.async_copy(h_src.at[srci_v.at[j]], rows_v, sem)

        def gwait(rows_v, sem):
            pltpu.make_async_copy(h_src.at[srci_v.at[0]], rows_v, sem).wait()

        def scatter(j, rows_v):
            pltpu.sync_copy(rows_v, acc_sh.at[dsti_v.at[j]], add=True)

        def run_phase(pbase, nch):
            pltpu.sync_copy(src_hbm.at[pl.ds(pbase, nch)],
                            srci_v.at[pl.ds(0, nch)])
            pltpu.sync_copy(dst_hbm.at[pl.ds(pbase, nch)],
                            dsti_v.at[pl.ds(0, nch)])
            gather(0, rows0_v, sem0)

            def step(i, carry):
                a = 2 * i
                gather(a + 1, rows1_v, sem1)
                gwait(rows0_v, sem0)
                scatter(a, rows0_v)
                gather(a + 2, rows0_v, sem0)
                gwait(rows1_v, sem1)
                scatter(a + 1, rows1_v)
                return carry

            lax.fori_loop(0, nch // 2, step, 0)
            gwait(rows0_v, sem0)
            if nch % 2 == 1:
                scatter(nch - 1, rows0_v)

        run_phase(base, PH0)
        run_phase(base + PH0, PH1)
        plsc.subcore_barrier()

        for t in range(SLAB // B):
            row0 = sid * SLAB + t * B
            pltpu.sync_copy(acc_sh.at[pl.ds(row0, B)], rows0_v)
            pltpu.sync_copy(rows0_v, out_hbm.at[cid, pl.ds(row0, B)])

    return prop


_prop64 = _make_prop(64, stage_h=True)

FH = 64
CH_TILE = NCHUNK_ALL = EPAD // B // NS
_PHASES = (40, 40, 40, 38)


@functools.partial(
    pl.kernel,
    out_type=jax.ShapeDtypeStruct((NC, NA, FH), jnp.float32),
    mesh=_mesh,
    scratch_types=[
        pltpu.VMEM((PH0, B), jnp.int32),
        pltpu.VMEM((IDXROWS, B), jnp.int32),
        pltpu.VMEM((B, FH), jnp.float32),
        pltpu.VMEM((B, FH), jnp.float32),
        pltpu.VMEM_SHARED((NA, FH), jnp.float32),
        pltpu.VMEM_SHARED((NN, FH), jnp.float32),
        pltpu.SemaphoreType.DMA,
        pltpu.SemaphoreType.DMA,
    ],
    compiler_params=pltpu.CompilerParams(use_tc_tiling_on_sc=False),
)
def _prop128(h3_hbm, src_hbm, dst_hbm, out_hbm, dsti_v, srci_v, rows0_v,
             rows1_v, acc_sh, h_sh, sem0, sem1):
    cid = lax.axis_index("c")
    sid = lax.axis_index("s")
    base = sid * CH_TILE

    zero16 = jnp.zeros((LANES,), jnp.float32)
    izero16 = jnp.zeros((LANES,), jnp.int32)

    hrows = NN // NS
    pltpu.sync_copy(h3_hbm.at[cid, pl.ds(sid * hrows, hrows)],
                    h_sh.at[pl.ds(sid * hrows, hrows)])

    def zb(r, carry):
        for f in range(FH // LANES):
            rows0_v[r, pl.ds(f * LANES, LANES)] = zero16
        return carry

    lax.fori_loop(0, B, zb, 0)
    for t in range(SLAB // B):
        pltpu.sync_copy(rows0_v, acc_sh.at[pl.ds(sid * SLAB + t * B, B)])
    for f in range(B // LANES):
        srci_v[PH0, pl.ds(f * LANES, LANES)] = izero16
    plsc.subcore_barrier()

    def gather(j, rows_v, sem):
        pltpu.async_copy(h_sh.at[srci_v.at[j]], rows_v, sem)

    def gwait(rows_v, sem):
        pltpu.make_async_copy(h_sh.at[srci_v.at[0]], rows_v, sem).wait()

    def scatter(j, rows_v):
        pltpu.sync_copy(rows_v, acc_sh.at[dsti_v.at[j]], add=True)

    def run_phase(pbase, nch):
        pltpu.sync_copy(src_hbm.at[pl.ds(pbase, nch)], srci_v.at[pl.ds(0, nch)])
        pltpu.sync_copy(dst_hbm.at[pl.ds(pbase, nch)], dsti_v.at[pl.ds(0, nch)])
        gather(0, rows0_v, sem0)

        def step(i, carry):
            a = 2 * i
            gather(a + 1, rows1_v, sem1)
            gwait(rows0_v, sem0)
            scatter(a, rows0_v)
            gather(a + 2, rows0_v, sem0)
            gwait(rows1_v, sem1)
            scatter(a + 1, rows1_v)
            return carry

        lax.fori_loop(0, nch // 2, step, 0)
        gwait(rows0_v, sem0)
        if nch % 2 == 1:
            scatter(nch - 1, rows0_v)

    off = 0
    for nch in _PHASES:
        run_phase(base + off, nch)
        off += nch
    plsc.subcore_barrier()

    for t in range(SLAB // B):
        row0 = sid * SLAB + t * B
        pltpu.sync_copy(acc_sh.at[pl.ds(row0, B)], rows0_v)
        pltpu.sync_copy(rows0_v, out_hbm.at[cid, pl.ds(row0, B)])


@functools.partial(
    pl.kernel,
    out_type=jax.ShapeDtypeStruct((NC, NA, DEGW), jnp.float32),
    mesh=_mesh,
    scratch_types=[
        pltpu.VMEM((CH_PER, B), jnp.int32),
        pltpu.VMEM((B, DEGW), jnp.float32),
        pltpu.VMEM((SLAB, DEGW), jnp.float32),
        pltpu.VMEM_SHARED((NA, DEGW), jnp.float32),
    ],
    compiler_params=pltpu.CompilerParams(use_tc_tiling_on_sc=False),
)
def _sc_degrees(dst_hbm, out_hbm, dsti_v, ones_v, zb_v, acc_sh):
    cid = lax.axis_index("c")
    sid = lax.axis_index("s")
    wid = sid * NC + cid

    one16 = jnp.ones((LANES,), jnp.float32)
    zero16 = jnp.zeros((LANES,), jnp.float32)

    def fill(r, carry):
        ones_v[r] = one16
        return carry

    lax.fori_loop(0, B, fill, 0)

    def zb(r, carry):
        zb_v[r] = zero16
        return carry

    lax.fori_loop(0, SLAB, zb, 0)
    pltpu.sync_copy(zb_v, acc_sh.at[pl.ds(sid * SLAB, SLAB)])
    pltpu.sync_copy(dst_hbm.at[pl.ds(wid * CH_PER, CH_PER)], dsti_v)
    plsc.subcore_barrier()

    def chunk(j, carry):
        pltpu.sync_copy(ones_v, acc_sh.at[dsti_v.at[j]], add=True)
        return carry

    lax.fori_loop(0, CH_PER, chunk, 0)
    plsc.subcore_barrier()

    row0 = sid * SLAB
    pltpu.sync_copy(acc_sh.at[pl.ds(row0, SLAB)], zb_v)
    pltpu.sync_copy(zb_v, out_hbm.at[cid, pl.ds(row0, SLAB)])


def _dot(a, b):
    return jnp.dot(a, b, preferred_element_type=jnp.float32)


def _row_spec(f):
    return pl.BlockSpec((BM, f), lambda i: (i, 0))


def _full_spec(shape):
    nd = len(shape)
    return pl.BlockSpec(shape, lambda i, _n=nd: (0,) * _n)


def _part_spec(f):
    return pl.BlockSpec((NC, BM, f), lambda i: (0, i, 0))


def _tc0_body(x_ref, w_ref, o_ref):
    o_ref[...] = _dot(x_ref[...], w_ref[...])


def _tc0(x, w):
    return pl.pallas_call(
        _tc0_body,
        grid=(GRID,),
        in_specs=[_row_spec(128), _full_spec((128, 128))],
        out_specs=_row_spec(128),
        out_shape=jax.ShapeDtypeStruct((NN, 128), jnp.float32),
    )(x, w)


def _tca_body(deg_ref, p1_ref, dis_ref, h1p_ref):
    d = deg_ref[...]
    dsum = d[0, :, 0:1] + d[1, :, 0:1] + 1.0
    dis = lax.rsqrt(dsum)
    dis_ref[...] = jnp.broadcast_to(dis, (BM, 128))
    h1p = dis * p1_ref[...]
    h1p_ref[0] = h1p[:, :FH]
    h1p_ref[1] = h1p[:, FH:]


def _tca(deg_p, p1):
    return pl.pallas_call(
        _tca_body,
        grid=(GRID,),
        in_specs=[_part_spec(DEGW), _row_spec(128)],
        out_specs=[_row_spec(128), _part_spec(FH)],
        out_shape=[jax.ShapeDtypeStruct((NN, 128), jnp.float32),
                   jax.ShapeDtypeStruct((NC, NN, FH), jnp.float32)],
    )(deg_p, p1)


def _tcb_body(dis_ref, s_ref, hp_ref, b_ref, w_ref, o_ref):
    dis = dis_ref[...]
    s = s_ref[...]
    hp = hp_ref[...]
    ah = jnp.concatenate([s[0] + hp[0], s[1] + hp[1]], axis=-1)
    conv = dis * ah + b_ref[...][None, :]
    h1 = jnp.maximum(conv, 0.0)
    o_ref[...] = dis[:, :64] * _dot(h1, w_ref[...])


def _tcb(dis, s1, h1p, b_e1, w_e2):
    return pl.pallas_call(
        _tcb_body,
        grid=(GRID,),
        in_specs=[_row_spec(128), _part_spec(FH), _part_spec(FH),
                  _full_spec((128,)), _full_spec((128, 64))],
        out_specs=_row_spec(64),
        out_shape=jax.ShapeDtypeStruct((NN, 64), jnp.float32),
    )(dis, s1, h1p, b_e1, w_e2)


def _tcc_body(dis_ref, s_ref, hp_ref, b2_ref, wfc_ref, bfc_ref, o_ref):
    dis = dis_ref[...][:, :64]
    s = s_ref[...]
    conv = dis * (s[0] + s[1] + hp_ref[...]) + b2_ref[...][None, :]
    z = _dot(conv, wfc_ref[...]) + bfc_ref[...][None, :]
    o_ref[...] = dis * z


def _tcc(dis, s2, h2p, b_e2, w_efc, b_efc):
    return pl.pallas_call(
        _tcc_body,
        grid=(GRID,),
        in_specs=[_row_spec(128), _part_spec(64), _row_spec(64),
                  _full_spec((64,)), _full_spec((64, 64)), _full_spec((64,))],
        out_specs=_row_spec(64),
        out_shape=jax.ShapeDtypeStruct((NN, 64), jnp.float32),
    )(dis, s2, h2p, b_e2, w_efc, b_efc)


def _tcd_body(dis_ref, s_ref, zp_ref, w1_ref, b1_ref, w2_ref, o_ref):
    dis = dis_ref[...]
    dis64 = dis[:, :64]
    s = s_ref[...]
    pz = dis64 * (s[0] + s[1] + zp_ref[...])
    h3 = jnp.maximum(_dot(pz, w1_ref[...]) + b1_ref[...][None, :], 0.0)
    gp = dis * _dot(h3, w2_ref[...])
    o_ref[0] = gp[:, :FH]
    o_ref[1] = gp[:, FH:]


def _tcd(dis, s3, zp, w_d1, b_d1, w_d2):
    return pl.pallas_call(
        _tcd_body,
        grid=(GRID,),
        in_specs=[_row_spec(128), _part_spec(64), _row_spec(64),
                  _full_spec((64, 256)), _full_spec((256,)),
                  _full_spec((256, 128))],
        out_specs=_part_spec(FH),
        out_shape=jax.ShapeDtypeStruct((NC, NN, FH), jnp.float32),
    )(dis, s3, zp, w_d1, b_d1, w_d2)


def _tce_body(dis_ref, s_ref, gp_ref, b2_ref, wfc_ref, bfc_ref, o_ref):
    dis = dis_ref[...]
    s = s_ref[...]
    gp = gp_ref[...]
    ah = jnp.concatenate([s[0] + gp[0], s[1] + gp[1]], axis=-1)
    h4 = dis * ah + b2_ref[...][None, :]
    o_ref[...] = _dot(h4, wfc_ref[...]) + bfc_ref[...][None, :]


def _tce(dis, s4, gp, b_d2, w_dfc, b_dfc):
    return pl.pallas_call(
        _tce_body,
        grid=(GRID,),
        in_specs=[_row_spec(128), _part_spec(FH), _part_spec(FH),
                  _full_spec((128,)), _full_spec((128, 1024)),
                  _full_spec((1024,))],
        out_specs=_row_spec(1024),
        out_shape=jax.ShapeDtypeStruct((NN, 1024), jnp.float32),
    )(dis, s4, gp, b_d2, w_dfc, b_dfc)


def kernel(x, edge_index, w_e1, b_e1, w_e2, b_e2, w_efc, b_efc,
           w_d1, b_d1, w_d2, b_d2, w_dfc, b_dfc):
    npad = EPAD - EE
    src = jnp.concatenate(
        [edge_index[0], jnp.zeros((npad,), jnp.int32)]).reshape(-1, B)
    dst = jnp.concatenate(
        [edge_index[1],
         NN + (jnp.arange(npad, dtype=jnp.int32) % 8)]).reshape(-1, B)

    deg_p = _sc_degrees(dst)
    p1 = _tc0(x, w_e1)
    dis, h1p = _tca(deg_p, p1)

    s1 = _prop128(h1p, src, dst)
    h2p = _tcb(dis, s1, h1p, b_e1, w_e2)

    s2 = _prop64(h2p, src, dst)
    zp = _tcc(dis, s2, h2p, b_e2, w_efc, b_efc)

    s3 = _prop64(zp, src, dst)
    gp = _tcd(dis, s3, zp, w_d1, b_d1, w_d2)

    s4 = _prop128(gp, src, dst)
    return _tce(dis, s4, gp, b_d2, w_dfc, b_dfc)
